# baseline (speedup 1.0000x reference)
import functools

import jax
import jax.numpy as jnp
from jax import lax
from jax.experimental import pallas as pl
from jax.experimental.pallas import tpu as pltpu
from jax.experimental.pallas import tpu_sc as plsc

N = 50000
D = 64
Q = 16
E = 800000
ET = E // 16
HC = 2000
HCH = ET // HC
NPAD = 50176
STRIPE = NPAD // 16
RCH = 224
L = 100000
L_PAD = 102400
LW = L_PAD // 32
LCH = LW // 128

_mesh = plsc.VectorSubcoreMesh(core_axis_name="c", subcore_axis_name="s",
                               num_cores=2, num_subcores=16)
_sc_params = pltpu.CompilerParams(use_tc_tiling_on_sc=False,
                                  needs_layout_passes=False)


def _nr_rsqrt(v):
    x = jnp.maximum(v, 1.0)
    i = lax.bitcast_convert_type(x, jnp.int32)
    i = jnp.int32(0x5F3759DF) - (i >> 1)
    y = lax.bitcast_convert_type(i, jnp.float32)
    for _ in range(3):
        y = y * (1.5 - 0.5 * x * y * y)
    return y


@functools.partial(
    pl.kernel,
    out_type=(tuple(jax.ShapeDtypeStruct((NPAD, 2 * Q), jnp.bfloat16) for _ in range(4))
              + (jax.ShapeDtypeStruct((NPAD,), jnp.float32),
                 jax.ShapeDtypeStruct((NPAD,), jnp.float32))),
    mesh=_mesh,
    scratch_types=[
        pltpu.VMEM((HC,), jnp.int32),
        pltpu.VMEM((HC,), jnp.float32),
        pltpu.VMEM((STRIPE,), jnp.float32),
        pltpu.VMEM((STRIPE,), jnp.float32),
        pltpu.VMEM((RCH, D), jnp.float32),
        pltpu.VMEM((RCH, 2 * Q), jnp.bfloat16),
        pltpu.VMEM((RCH, 2 * Q), jnp.bfloat16),
        pltpu.VMEM_SHARED((NPAD,), jnp.float32),
        pltpu.VMEM_SHARED((NPAD,), jnp.float32),
    ],
    compiler_params=_sc_params,
)
def _prep_kernel(e_ui, e_iu, xu, xi,
                 tul, tuh, til, tih, rs_du, rs_di,
                 idx_v, ones_v, deg_v, rs_v, x_v, qbl, qbh,
                 sh_a, sh_b):
    c = lax.axis_index("c")
    s = lax.axis_index("s")

    def fill_ones(i, carry):
        ones_v[pl.ds(i * 16, 16)] = jnp.ones((16,), jnp.float32)
        return carry
    lax.fori_loop(0, HC // 16, fill_ones, 0)

    def fill_zer(i, carry):
        rs_v[pl.ds(i * 16, 16)] = jnp.zeros((16,), jnp.float32)
        return carry
    lax.fori_loop(0, STRIPE // 16, fill_zer, 0)

    sl = pl.ds(s * STRIPE, STRIPE)
    pltpu.sync_copy(rs_v, sh_a.at[sl])
    pltpu.sync_copy(rs_v, sh_b.at[sl])
    plsc.subcore_barrier()

    def hist(arr, row, sh):
        def chunk_body(j, carry):
            base = s * ET + j * HC
            pltpu.sync_copy(arr.at[row, pl.ds(base, HC)], idx_v)
            pltpu.sync_copy(ones_v, sh.at[idx_v], add=True)
            return carry
        lax.fori_loop(0, HCH, chunk_body, 0)

    @pl.when(c == 0)
    def _():
        hist(e_ui, 0, sh_a)
        hist(e_ui, 1, sh_b)

    @pl.when(c == 1)
    def _():
        hist(e_iu, 0, sh_a)
        hist(e_iu, 1, sh_b)

    plsc.subcore_barrier()

    def rs_from(sh):
        pltpu.sync_copy(sh.at[sl], deg_v)

        def body(i, carry):
            rs_v[pl.ds(i * 16, 16)] = _nr_rsqrt(deg_v[pl.ds(i * 16, 16)])
            return carry
        lax.fori_loop(0, STRIPE // 16, body, 0)

    rs_from(sh_b)

    @pl.when(c == 0)
    def _():
        pltpu.sync_copy(rs_v, rs_du.at[sl])

    @pl.when(c == 1)
    def _():
        pltpu.sync_copy(rs_v, rs_di.at[sl])

    rs_from(sh_a)

    def prescale(x, ql, qh):
        for t in range(STRIPE // RCH):
            base = s * STRIPE + t * RCH
            off = jnp.minimum(base, N - RCH)
            pltpu.sync_copy(x.at[pl.ds(off, RCH)], x_v)
            rbase = off - s * STRIPE

            def grp_body(g, carry):
                rsvec = rs_v[pl.ds(rbase + g * 16, 16)]
                for rr in range(16):
                    r = g * 16 + rr
                    rsc = rsvec[rr]
                    qbl[r, :] = plsc.pack(x_v[r, pl.ds(0, Q)] * rsc,
                                          x_v[r, pl.ds(Q, Q)] * rsc,
                                          format=plsc.PackFormat.INTERLEAVED)
                    qbh[r, :] = plsc.pack(x_v[r, pl.ds(2 * Q, Q)] * rsc,
                                          x_v[r, pl.ds(3 * Q, Q)] * rsc,
                                          format=plsc.PackFormat.INTERLEAVED)
                return carry
            lax.fori_loop(0, RCH // 16, grp_body, 0)
            osl = pl.ds(off, RCH)
            pltpu.sync_copy(qbl, ql.at[osl])
            pltpu.sync_copy(qbh, qh.at[osl])

    @pl.when(c == 0)
    def _():
        prescale(xu, tul, tuh)

    @pl.when(c == 1)
    def _():
        prescale(xi, til, tih)


H = 2 * Q
K2C = 200
K2G = 10
K2GR = ET // (K2C * K2G)


@functools.partial(
    pl.kernel,
    out_type=(jax.ShapeDtypeStruct((N, D), jnp.float32),
              jax.ShapeDtypeStruct((N, D), jnp.float32)),
    mesh=_mesh,
    scratch_types=[
        pltpu.VMEM((K2C * K2G,), jnp.int32),
        pltpu.VMEM((K2C * K2G,), jnp.int32),
        pltpu.VMEM((K2C, H), jnp.bfloat16),
        pltpu.VMEM((K2C, H), jnp.bfloat16),
        pltpu.VMEM((K2C, H), jnp.float32),
        pltpu.VMEM((K2C, H), jnp.float32),
        pltpu.VMEM((STRIPE // 32, H), jnp.float32),
        pltpu.VMEM((STRIPE // 32, H), jnp.float32),
        pltpu.VMEM_SHARED((NPAD, H), jnp.float32),
        pltpu.SemaphoreType.DMA,
        pltpu.SemaphoreType.DMA,
        pltpu.SemaphoreType.DMA,
        pltpu.SemaphoreType.DMA,
    ],
    compiler_params=_sc_params,
)
def _segsum_kernel(tul, tuh, til, tih,
                   e_ui, e_iu,
                   a_ui, a_iu,
                   idxs, idxd, rows0, rows1, frows0, frows1, zer_v, bnc_v, sh,
                   semg0, semg1, sems0, sems1):
    c = lax.axis_index("c")
    s = lax.axis_index("s")

    def fill_zer(i, carry):
        zer_v[i, pl.ds(0, 16)] = jnp.zeros((16,), jnp.float32)
        zer_v[i, pl.ds(16, 16)] = jnp.zeros((16,), jnp.float32)
        return carry
    lax.fori_loop(0, STRIPE // 32, fill_zer, 0)

    def zero_shared():
        for t in range(32):
            pltpu.sync_copy(zer_v, sh.at[pl.ds(s * STRIPE + t * (STRIPE // 32),
                                               STRIPE // 32)])

    rows = (rows0, rows1)
    frows = (frows0, frows1)
    semg = (semg0, semg1)
    sems = (sems0, sems1)

    def unpack_rows(src_b, dst_f):
        def rbody(r, carry):
            v0, v1 = plsc.unpack(src_b[r, :], format=plsc.PackFormat.INTERLEAVED)
            dst_f[r, pl.ds(0, Q)] = v0
            dst_f[r, pl.ds(Q, Q)] = v1
            return carry
        lax.fori_loop(0, K2C, rbody, 0)

    def accumulate(tab, earr):
        def group_body(g, carry):
            base = s * ET + g * (K2C * K2G)
            pltpu.sync_copy(earr.at[0, pl.ds(base, K2C * K2G)], idxs)
            pltpu.sync_copy(earr.at[1, pl.ds(base, K2C * K2G)], idxd)
            gd = [None] * K2G
            sd = [None] * K2G
            for k in range(K2G):
                sl = k % 2
                if k >= 2:
                    sd[k - 2].wait()
                gd[k] = pltpu.async_copy(
                    tab.at[idxs.at[pl.ds(k * K2C, K2C)]], rows[sl], semg[sl])
                if k >= 1:
                    psl = (k - 1) % 2
                    gd[k - 1].wait()
                    unpack_rows(rows[psl], frows[psl])
                    sd[k - 1] = pltpu.async_copy(
                        frows[psl], sh.at[idxd.at[pl.ds((k - 1) * K2C, K2C)]],
                        sems[psl], add=True)
            gd[K2G - 1].wait()
            lsl = (K2G - 1) % 2
            unpack_rows(rows[lsl], frows[lsl])
            sd[K2G - 1] = pltpu.async_copy(
                frows[lsl], sh.at[idxd.at[pl.ds((K2G - 1) * K2C, K2C)]],
                sems[lsl], add=True)
            sd[K2G - 2].wait()
            sd[K2G - 1].wait()
            return carry
        lax.fori_loop(0, K2GR, group_body, 0)

    def writeout(out, half):
        for t in range(32):
            off = jnp.minimum(s * STRIPE + t * (STRIPE // 32), N - STRIPE // 32)
            rsl = pl.ds(off, STRIPE // 32)
            pltpu.sync_copy(sh.at[rsl], bnc_v)
            pltpu.sync_copy(bnc_v, out.at[rsl, pl.ds(half * H, H)])

    phases = (
        ((tul, e_ui, a_ui), (tuh, e_ui, a_ui)),
        ((til, e_iu, a_iu), (tih, e_iu, a_iu)),
    )
    for (tb0, e0, o0), (tb1, e1, o1) in phases:
        zero_shared()
        plsc.subcore_barrier()

        @pl.when(c == 0)
        def _():
            accumulate(tb0, e0)

        @pl.when(c == 1)
        def _():
            accumulate(tb1, e1)

        plsc.subcore_barrier()

        @pl.when(c == 0)
        def _():
            writeout(o0, 0)

        @pl.when(c == 1)
        def _():
            writeout(o1, 1)

        plsc.subcore_barrier()


_BLK = 2000


def _combine_body(xu_ref, xi_ref, aui_ref, aiu_ref, rdu_ref, rdi_ref,
                  wlu_t, wiu_t, wli_t, wii_t,
                  xcu_ref, xci_ref):
    def cell(xd, a_ref, rs, wl_t, wi_t):
        a = a_ref[...] * rs
        z = (jnp.dot(xd + a, wl_t, preferred_element_type=jnp.float32)
             + jnp.dot(xd * a, wi_t, preferred_element_type=jnp.float32))
        return jnp.where(z >= 0, z, 0.01 * z)

    xi_ = xi_ref[...]
    xu_ = xu_ref[...]
    xci_ref[...] = jnp.concatenate(
        [xi_, cell(xi_, aui_ref, rdu_ref[...], wlu_t[...], wiu_t[...])], axis=-1)
    xcu_ref[...] = jnp.concatenate(
        [xu_, cell(xu_, aiu_ref, rdi_ref[...], wli_t[...], wii_t[...])], axis=-1)


def _combine(x_u, x_i, a_ui, a_iu, rdu, rdi, wlu_t, wiu_t, wli_t, wii_t):
    grid = (N // _BLK,)
    row = pl.BlockSpec((_BLK, D), lambda i: (i, 0))
    col = pl.BlockSpec((_BLK, 1), lambda i: (i, 0))
    wide = pl.BlockSpec((_BLK, 2 * D), lambda i: (i, 0))
    wspec = pl.BlockSpec((D, D), lambda i: (0, 0))
    return pl.pallas_call(
        _combine_body,
        grid=grid,
        in_specs=[row, row, row, row, col, col, wspec, wspec, wspec, wspec],
        out_specs=[wide, wide],
        out_shape=(jax.ShapeDtypeStruct((N, 2 * D), jnp.float32),
                   jax.ShapeDtypeStruct((N, 2 * D), jnp.float32)),
    )(x_u, x_i, a_ui, a_iu, rdu, rdi, wlu_t, wiu_t, wli_t, wii_t)


@functools.partial(
    pl.kernel,
    out_type=jax.ShapeDtypeStruct((L_PAD,), jnp.float32),
    mesh=_mesh,
    scratch_types=[
        pltpu.VMEM((LW,), jnp.int32),
        pltpu.VMEM((LW,), jnp.int32),
        pltpu.VMEM((128, 2 * D), jnp.float32),
        pltpu.VMEM((128, 2 * D), jnp.float32),
        pltpu.VMEM((128, 2 * D), jnp.float32),
        pltpu.VMEM((128, 2 * D), jnp.float32),
        pltpu.VMEM((LW,), jnp.float32),
        pltpu.SemaphoreType.DMA,
        pltpu.SemaphoreType.DMA,
    ],
    compiler_params=_sc_params,
)
def _label_kernel(xcu, xci, l0, l1, y,
                  l0_v, l1_v, a_b0, b_b0, a_b1, b_b1, y_b, sem0, sem1):
    c = lax.axis_index("c")
    s = lax.axis_index("s")
    w = c * 16 + s

    pltpu.sync_copy(l0.at[pl.ds(w * LW, LW)], l0_v)
    pltpu.sync_copy(l1.at[pl.ds(w * LW, LW)], l1_v)

    lanes = lax.iota(jnp.int32, 16)

    def dots(j, a_b, b_b):
        def group_body(g, carry2):
            def lane_body(rr, vec):
                r = g * 16 + rr
                acc = a_b[r, pl.ds(0, 16)] * b_b[r, pl.ds(0, 16)]
                for q in range(1, 8):
                    acc = acc + a_b[r, pl.ds(q * 16, 16)] * b_b[r, pl.ds(q * 16, 16)]
                return jnp.where(lanes == rr, jnp.sum(acc), vec)
            vec = lax.fori_loop(0, 16, lane_body, jnp.zeros((16,), jnp.float32))
            y_b[pl.ds(j * 128 + g * 16, 16)] = vec
            return carry2
        lax.fori_loop(0, 8, group_body, 0)

    def fire(j, a_b, b_b, sem):
        sl = pl.ds(j * 128, 128)
        return (pltpu.async_copy(xcu.at[l0_v.at[sl]], a_b, sem),
                pltpu.async_copy(xci.at[l1_v.at[sl]], b_b, sem))

    def pair_body(p, carry):
        j0 = 2 * p
        da = fire(j0, a_b0, b_b0, sem0)
        db = fire(j0 + 1, a_b1, b_b1, sem1)
        for d_ in da:
            d_.wait()
        dots(j0, a_b0, b_b0)
        for d_ in db:
            d_.wait()
        dots(j0 + 1, a_b1, b_b1)
        return carry
    lax.fori_loop(0, LCH // 2, pair_body, 0)

    dt = fire(LCH - 1, a_b0, b_b0, sem0)
    for d_ in dt:
        d_.wait()
    dots(LCH - 1, a_b0, b_b0)

    pltpu.sync_copy(y_b, y.at[pl.ds(w * LW, LW)])


def kernel(n_id_user, n_id_item, edge_index_ui, edge_index_iu, edge_label_index,
           emb_user, emb_item, W_loop_ui, W_intr_ui, W_loop_iu, W_intr_iu):
    del n_id_user, n_id_item
    f32 = jnp.float32
    i32 = jnp.int32
    x_u = emb_user.astype(f32)
    x_i = emb_item.astype(f32)

    e_ui = edge_index_ui.astype(i32)
    e_iu = edge_index_iu.astype(i32)

    tul, tuh, til, tih, rs_du, rs_di = _prep_kernel(e_ui, e_iu, x_u, x_i)

    a_ui, a_iu = _segsum_kernel(tul, tuh, til, tih, e_ui, e_iu)

    xcat_u, xcat_i = _combine(
        x_u, x_i, a_ui, a_iu,
        rs_du[:N].reshape(N, 1), rs_di[:N].reshape(N, 1),
        W_loop_ui.T, W_intr_ui.T, W_loop_iu.T, W_intr_iu.T)

    lpad = jnp.zeros((L_PAD - L,), i32)
    l0 = jnp.concatenate([edge_label_index[0].astype(i32), lpad])
    l1 = jnp.concatenate([edge_label_index[1].astype(i32), lpad])
    y = _label_kernel(xcat_u, xcat_i, l0, l1)
    return y[:L]


# speedup vs baseline: 1.1864x; 1.1864x over previous
import functools

import jax
import jax.numpy as jnp
from jax import lax
from jax.experimental import pallas as pl
from jax.experimental.pallas import tpu as pltpu
from jax.experimental.pallas import tpu_sc as plsc

N = 50000
D = 64
Q = 16
E = 800000
ET = E // 16
HC = 2000
HCH = ET // HC
NPAD = 50176
STRIPE = NPAD // 16
RCH = 224
L = 100000
L_PAD = 102400
LW = L_PAD // 32
LCH = LW // 128

_mesh = plsc.VectorSubcoreMesh(core_axis_name="c", subcore_axis_name="s",
                               num_cores=2, num_subcores=16)
_sc_params = pltpu.CompilerParams(use_tc_tiling_on_sc=False,
                                  needs_layout_passes=False)


def _nr_rsqrt(v):
    x = jnp.maximum(v, 1.0)
    i = lax.bitcast_convert_type(x, jnp.int32)
    i = jnp.int32(0x5F3759DF) - (i >> 1)
    y = lax.bitcast_convert_type(i, jnp.float32)
    for _ in range(3):
        y = y * (1.5 - 0.5 * x * y * y)
    return y


@functools.partial(
    pl.kernel,
    out_type=(tuple(jax.ShapeDtypeStruct((NPAD, 2 * Q), jnp.float32) for _ in range(4))
              + (jax.ShapeDtypeStruct((NPAD,), jnp.float32),
                 jax.ShapeDtypeStruct((NPAD,), jnp.float32))),
    mesh=_mesh,
    scratch_types=[
        pltpu.VMEM((HC,), jnp.int32),
        pltpu.VMEM((HC,), jnp.int32),
        pltpu.VMEM((NPAD // 2,), jnp.float32),
        pltpu.VMEM((NPAD // 2,), jnp.float32),
        pltpu.VMEM((STRIPE,), jnp.float32),
        pltpu.VMEM((STRIPE,), jnp.float32),
        pltpu.VMEM((RCH, D), jnp.float32),
        pltpu.VMEM((RCH, 2 * Q), jnp.float32),
        pltpu.VMEM((RCH, 2 * Q), jnp.float32),
        pltpu.VMEM_SHARED((NPAD,), jnp.float32),
        pltpu.VMEM_SHARED((NPAD,), jnp.float32),
        pltpu.VMEM_SHARED((16, NPAD // 2), jnp.float32),
    ],
    compiler_params=_sc_params,
)
def _prep_kernel(e_ui, e_iu, xu, xi,
                 tul, tuh, til, tih, rs_du, rs_di,
                 idxa_v, idxb_v, lh_a, lh_b, deg_v, rs_v, x_v, qbl, qbh,
                 sh_a, sh_b, stage):
    c = lax.axis_index("c")
    s = lax.axis_index("s")

    sl = pl.ds(s * STRIPE, STRIPE)
    NH = NPAD // 2
    ones16 = jnp.ones((16,), jnp.float32)

    def build_half(earr, half):
        hbase = half * NH

        def zero_lh(i, carry):
            lh_a[pl.ds(i * 16, 16)] = jnp.zeros((16,), jnp.float32)
            lh_b[pl.ds(i * 16, 16)] = jnp.zeros((16,), jnp.float32)
            return carry
        lax.fori_loop(0, NH // 16, zero_lh, 0)

        def chunk_body(j, carry):
            base = s * ET + j * HC
            pltpu.sync_copy(earr.at[0, pl.ds(base, HC)], idxa_v)
            pltpu.sync_copy(earr.at[1, pl.ds(base, HC)], idxb_v)

            def vec_body(i, carry2):
                iva = idxa_v[pl.ds(i * 16, 16)] - hbase
                ivb = idxb_v[pl.ds(i * 16, 16)] - hbase
                ma = (iva >= 0) & (iva < NH)
                mb = (ivb >= 0) & (ivb < NH)
                plsc.addupdate_scatter(lh_a, [iva], ones16, mask=ma)
                plsc.addupdate_scatter(lh_b, [ivb], ones16, mask=mb)
                return carry2
            lax.fori_loop(0, HC // 16, vec_body, 0)
            return carry
        lax.fori_loop(0, HCH, chunk_body, 0)

        seg = NH // 16
        acc_sl = pl.ds(s * seg, seg)
        for lh, sh in ((lh_a, sh_a), (lh_b, sh_b)):
            pltpu.sync_copy(lh, stage.at[s])
            plsc.subcore_barrier()
            pltpu.sync_copy(stage.at[0, acc_sl], deg_v.at[pl.ds(0, seg)])

            def t_body(t, carry):
                pltpu.sync_copy(stage.at[t, acc_sl], deg_v.at[pl.ds(seg, seg)])

                def add_body(i, carry2):
                    deg_v[pl.ds(i * 16, 16)] = (deg_v[pl.ds(i * 16, 16)]
                                                + deg_v[pl.ds(seg + i * 16, 16)])
                    return carry2
                lax.fori_loop(0, seg // 16, add_body, 0)
                return carry
            lax.fori_loop(1, 16, t_body, 0)
            pltpu.sync_copy(deg_v.at[pl.ds(0, seg)],
                            sh.at[pl.ds(hbase + s * seg, seg)])
            plsc.subcore_barrier()

    @pl.when(c == 0)
    def _():
        build_half(e_ui, 0)
        build_half(e_ui, 1)

    @pl.when(c == 1)
    def _():
        build_half(e_iu, 0)
        build_half(e_iu, 1)

    plsc.subcore_barrier()

    def rs_from(sh):
        pltpu.sync_copy(sh.at[sl], deg_v)

        def body(i, carry):
            rs_v[pl.ds(i * 16, 16)] = _nr_rsqrt(deg_v[pl.ds(i * 16, 16)])
            return carry
        lax.fori_loop(0, STRIPE // 16, body, 0)

    rs_from(sh_b)

    @pl.when(c == 0)
    def _():
        pltpu.sync_copy(rs_v, rs_du.at[sl])

    @pl.when(c == 1)
    def _():
        pltpu.sync_copy(rs_v, rs_di.at[sl])

    rs_from(sh_a)

    def prescale(x, ql, qh):
        for t in range(STRIPE // RCH):
            base = s * STRIPE + t * RCH
            off = jnp.minimum(base, N - RCH)
            pltpu.sync_copy(x.at[pl.ds(off, RCH)], x_v)
            rbase = off - s * STRIPE

            def grp_body(g, carry):
                rsvec = rs_v[pl.ds(rbase + g * 16, 16)]
                for rr in range(16):
                    r = g * 16 + rr
                    rsc = rsvec[rr]
                    qbl[r, pl.ds(0, Q)] = x_v[r, pl.ds(0, Q)] * rsc
                    qbl[r, pl.ds(Q, Q)] = x_v[r, pl.ds(Q, Q)] * rsc
                    qbh[r, pl.ds(0, Q)] = x_v[r, pl.ds(2 * Q, Q)] * rsc
                    qbh[r, pl.ds(Q, Q)] = x_v[r, pl.ds(3 * Q, Q)] * rsc
                return carry
            lax.fori_loop(0, RCH // 16, grp_body, 0)
            osl = pl.ds(off, RCH)
            pltpu.sync_copy(qbl, ql.at[osl])
            pltpu.sync_copy(qbh, qh.at[osl])

    @pl.when(c == 0)
    def _():
        prescale(xu, tul, tuh)

    @pl.when(c == 1)
    def _():
        prescale(xi, til, tih)


H = 2 * Q
K2C = 200
K2G = 25
K2GR = ET // (K2C * K2G)


@functools.partial(
    pl.kernel,
    out_type=(jax.ShapeDtypeStruct((N, D), jnp.float32),
              jax.ShapeDtypeStruct((N, D), jnp.float32)),
    mesh=_mesh,
    scratch_types=[
        pltpu.VMEM((K2C * K2G,), jnp.int32),
        pltpu.VMEM((K2C * K2G,), jnp.int32),
        pltpu.VMEM((K2C, H), jnp.float32),
        pltpu.VMEM((K2C, H), jnp.float32),
        pltpu.VMEM((STRIPE // 32, H), jnp.float32),
        pltpu.VMEM((STRIPE // 32, H), jnp.float32),
        pltpu.VMEM_SHARED((NPAD, H), jnp.float32),
        pltpu.SemaphoreType.DMA,
        pltpu.SemaphoreType.DMA,
        pltpu.SemaphoreType.DMA,
        pltpu.SemaphoreType.DMA,
    ],
    compiler_params=_sc_params,
)
def _segsum_kernel(tul, tuh, til, tih,
                   e_ui, e_iu,
                   a_ui, a_iu,
                   idxs, idxd, rows0, rows1, zer_v, bnc_v, sh,
                   semg0, semg1, sems0, sems1):
    c = lax.axis_index("c")
    s = lax.axis_index("s")

    def fill_zer(i, carry):
        zer_v[i, pl.ds(0, 16)] = jnp.zeros((16,), jnp.float32)
        zer_v[i, pl.ds(16, 16)] = jnp.zeros((16,), jnp.float32)
        return carry
    lax.fori_loop(0, STRIPE // 32, fill_zer, 0)

    def zero_shared():
        for t in range(32):
            pltpu.sync_copy(zer_v, sh.at[pl.ds(s * STRIPE + t * (STRIPE // 32),
                                               STRIPE // 32)])

    rows = (rows0, rows1)
    semg = (semg0, semg1)
    sems = (sems0, sems1)

    def accumulate(tab, earr):
        def group_body(g, carry):
            base = s * ET + g * (K2C * K2G)
            pltpu.sync_copy(earr.at[0, pl.ds(base, K2C * K2G)], idxs)
            pltpu.sync_copy(earr.at[1, pl.ds(base, K2C * K2G)], idxd)
            gd = [None] * K2G
            sd = [None] * K2G
            for k in range(K2G):
                sl = k % 2
                if k >= 2:
                    sd[k - 2].wait()
                gd[k] = pltpu.async_copy(
                    tab.at[idxs.at[pl.ds(k * K2C, K2C)]], rows[sl], semg[sl])
                if k >= 1:
                    psl = (k - 1) % 2
                    gd[k - 1].wait()
                    sd[k - 1] = pltpu.async_copy(
                        rows[psl], sh.at[idxd.at[pl.ds((k - 1) * K2C, K2C)]],
                        sems[psl], add=True)
            gd[K2G - 1].wait()
            lsl = (K2G - 1) % 2
            sd[K2G - 1] = pltpu.async_copy(
                rows[lsl], sh.at[idxd.at[pl.ds((K2G - 1) * K2C, K2C)]],
                sems[lsl], add=True)
            sd[K2G - 2].wait()
            sd[K2G - 1].wait()
            return carry
        lax.fori_loop(0, K2GR, group_body, 0)

    def writeout(out, half):
        for t in range(32):
            off = jnp.minimum(s * STRIPE + t * (STRIPE // 32), N - STRIPE // 32)
            rsl = pl.ds(off, STRIPE // 32)
            pltpu.sync_copy(sh.at[rsl], bnc_v)
            pltpu.sync_copy(bnc_v, out.at[rsl, pl.ds(half * H, H)])

    phases = (
        ((tul, e_ui, a_ui), (tuh, e_ui, a_ui)),
        ((til, e_iu, a_iu), (tih, e_iu, a_iu)),
    )
    for (tb0, e0, o0), (tb1, e1, o1) in phases:
        zero_shared()
        plsc.subcore_barrier()

        @pl.when(c == 0)
        def _():
            accumulate(tb0, e0)

        @pl.when(c == 1)
        def _():
            accumulate(tb1, e1)

        plsc.subcore_barrier()

        @pl.when(c == 0)
        def _():
            writeout(o0, 0)

        @pl.when(c == 1)
        def _():
            writeout(o1, 1)

        plsc.subcore_barrier()


_BLK = 2000


def _combine_body(xu_ref, xi_ref, aui_ref, aiu_ref, rdu_ref, rdi_ref,
                  wlu_t, wiu_t, wli_t, wii_t,
                  xcu_ref, xci_ref):
    def cell(xd, a_ref, rs, wl_t, wi_t):
        a = a_ref[...] * rs
        z = (jnp.dot(xd + a, wl_t, preferred_element_type=jnp.float32)
             + jnp.dot(xd * a, wi_t, preferred_element_type=jnp.float32))
        return jnp.where(z >= 0, z, 0.01 * z)

    xi_ = xi_ref[...]
    xu_ = xu_ref[...]
    xci_ref[...] = jnp.concatenate(
        [xi_, cell(xi_, aui_ref, rdu_ref[...], wlu_t[...], wiu_t[...])], axis=-1)
    xcu_ref[...] = jnp.concatenate(
        [xu_, cell(xu_, aiu_ref, rdi_ref[...], wli_t[...], wii_t[...])], axis=-1)


def _combine(x_u, x_i, a_ui, a_iu, rdu, rdi, wlu_t, wiu_t, wli_t, wii_t):
    grid = (N // _BLK,)
    row = pl.BlockSpec((_BLK, D), lambda i: (i, 0))
    col = pl.BlockSpec((_BLK, 1), lambda i: (i, 0))
    wide = pl.BlockSpec((_BLK, 2 * D), lambda i: (i, 0))
    wspec = pl.BlockSpec((D, D), lambda i: (0, 0))
    return pl.pallas_call(
        _combine_body,
        grid=grid,
        in_specs=[row, row, row, row, col, col, wspec, wspec, wspec, wspec],
        out_specs=[wide, wide],
        out_shape=(jax.ShapeDtypeStruct((N, 2 * D), jnp.float32),
                   jax.ShapeDtypeStruct((N, 2 * D), jnp.float32)),
    )(x_u, x_i, a_ui, a_iu, rdu, rdi, wlu_t, wiu_t, wli_t, wii_t)


@functools.partial(
    pl.kernel,
    out_type=jax.ShapeDtypeStruct((L_PAD,), jnp.float32),
    mesh=_mesh,
    scratch_types=[
        pltpu.VMEM((LW,), jnp.int32),
        pltpu.VMEM((LW,), jnp.int32),
        pltpu.VMEM((128, 2 * D), jnp.float32),
        pltpu.VMEM((128, 2 * D), jnp.float32),
        pltpu.VMEM((128, 2 * D), jnp.float32),
        pltpu.VMEM((128, 2 * D), jnp.float32),
        pltpu.VMEM((LW,), jnp.float32),
        pltpu.SemaphoreType.DMA,
        pltpu.SemaphoreType.DMA,
    ],
    compiler_params=_sc_params,
)
def _label_kernel(xcu, xci, l0, l1, y,
                  l0_v, l1_v, a_b0, b_b0, a_b1, b_b1, y_b, sem0, sem1):
    c = lax.axis_index("c")
    s = lax.axis_index("s")
    w = c * 16 + s

    pltpu.sync_copy(l0.at[pl.ds(w * LW, LW)], l0_v)
    pltpu.sync_copy(l1.at[pl.ds(w * LW, LW)], l1_v)

    lanes = lax.iota(jnp.int32, 16)

    def dots(j, a_b, b_b):
        def group_body(g, carry2):
            def lane_body(rr, vec):
                r = g * 16 + rr
                acc = a_b[r, pl.ds(0, 16)] * b_b[r, pl.ds(0, 16)]
                for q in range(1, 8):
                    acc = acc + a_b[r, pl.ds(q * 16, 16)] * b_b[r, pl.ds(q * 16, 16)]
                return jnp.where(lanes == rr, jnp.sum(acc), vec)
            vec = lax.fori_loop(0, 16, lane_body, jnp.zeros((16,), jnp.float32))
            y_b[pl.ds(j * 128 + g * 16, 16)] = vec
            return carry2
        lax.fori_loop(0, 8, group_body, 0)

    def fire(j, a_b, b_b, sem):
        sl = pl.ds(j * 128, 128)
        return (pltpu.async_copy(xcu.at[l0_v.at[sl]], a_b, sem),
                pltpu.async_copy(xci.at[l1_v.at[sl]], b_b, sem))

    def pair_body(p, carry):
        j0 = 2 * p
        da = fire(j0, a_b0, b_b0, sem0)
        db = fire(j0 + 1, a_b1, b_b1, sem1)
        for d_ in da:
            d_.wait()
        dots(j0, a_b0, b_b0)
        for d_ in db:
            d_.wait()
        dots(j0 + 1, a_b1, b_b1)
        return carry
    lax.fori_loop(0, LCH // 2, pair_body, 0)

    dt = fire(LCH - 1, a_b0, b_b0, sem0)
    for d_ in dt:
        d_.wait()
    dots(LCH - 1, a_b0, b_b0)

    pltpu.sync_copy(y_b, y.at[pl.ds(w * LW, LW)])


def kernel(n_id_user, n_id_item, edge_index_ui, edge_index_iu, edge_label_index,
           emb_user, emb_item, W_loop_ui, W_intr_ui, W_loop_iu, W_intr_iu):
    del n_id_user, n_id_item
    f32 = jnp.float32
    i32 = jnp.int32
    x_u = emb_user.astype(f32)
    x_i = emb_item.astype(f32)

    e_ui = edge_index_ui.astype(i32)
    e_iu = edge_index_iu.astype(i32)

    tul, tuh, til, tih, rs_du, rs_di = _prep_kernel(e_ui, e_iu, x_u, x_i)

    a_ui, a_iu = _segsum_kernel(tul, tuh, til, tih, e_ui, e_iu)

    xcat_u, xcat_i = _combine(
        x_u, x_i, a_ui, a_iu,
        rs_du[:N].reshape(N, 1), rs_di[:N].reshape(N, 1),
        W_loop_ui.T, W_intr_ui.T, W_loop_iu.T, W_intr_iu.T)

    lpad = jnp.zeros((L_PAD - L,), i32)
    l0 = jnp.concatenate([edge_label_index[0].astype(i32), lpad])
    l1 = jnp.concatenate([edge_label_index[1].astype(i32), lpad])
    y = _label_kernel(xcat_u, xcat_i, l0, l1)
    return y[:L]


# speedup vs baseline: 1.3067x; 1.1014x over previous
import functools

import jax
import jax.numpy as jnp
from jax import lax
from jax.experimental import pallas as pl
from jax.experimental.pallas import tpu as pltpu
from jax.experimental.pallas import tpu_sc as plsc

N = 50000
D = 64
Q = 16
E = 800000
ET = E // 16
HC = 2000
HCH = ET // HC
NPAD = 50176
STRIPE = NPAD // 16
RCH = 224
L = 100000
L_PAD = 102400
LW = L_PAD // 32
LCH = LW // 128

_mesh = plsc.VectorSubcoreMesh(core_axis_name="c", subcore_axis_name="s",
                               num_cores=2, num_subcores=16)
_sc_params = pltpu.CompilerParams(use_tc_tiling_on_sc=False,
                                  needs_layout_passes=False)


def _nr_rsqrt(v):
    x = jnp.maximum(v, 1.0)
    i = lax.bitcast_convert_type(x, jnp.int32)
    i = jnp.int32(0x5F3759DF) - (i >> 1)
    y = lax.bitcast_convert_type(i, jnp.float32)
    for _ in range(3):
        y = y * (1.5 - 0.5 * x * y * y)
    return y


@functools.partial(
    pl.kernel,
    out_type=(tuple(jax.ShapeDtypeStruct((NPAD, 2 * Q), jnp.float32) for _ in range(4))
              + (jax.ShapeDtypeStruct((NPAD,), jnp.float32),
                 jax.ShapeDtypeStruct((NPAD,), jnp.float32))),
    mesh=_mesh,
    scratch_types=[
        pltpu.VMEM((HC,), jnp.int32),
        pltpu.VMEM((HC,), jnp.float32),
        pltpu.VMEM((STRIPE,), jnp.float32),
        pltpu.VMEM((STRIPE,), jnp.float32),
        pltpu.VMEM((RCH, D), jnp.float32),
        pltpu.VMEM((RCH, 2 * Q), jnp.float32),
        pltpu.VMEM((RCH, 2 * Q), jnp.float32),
        pltpu.VMEM_SHARED((NPAD,), jnp.float32),
        pltpu.VMEM_SHARED((NPAD,), jnp.float32),
    ],
    compiler_params=_sc_params,
)
def _prep_kernel(e_ui, e_iu, xu, xi,
                 tul, tuh, til, tih, rs_du, rs_di,
                 idx_v, ones_v, deg_v, rs_v, x_v, qbl, qbh,
                 sh_a, sh_b):
    c = lax.axis_index("c")
    s = lax.axis_index("s")

    def fill_ones(i, carry):
        ones_v[pl.ds(i * 16, 16)] = jnp.ones((16,), jnp.float32)
        return carry
    lax.fori_loop(0, HC // 16, fill_ones, 0)

    def fill_zer(i, carry):
        rs_v[pl.ds(i * 16, 16)] = jnp.zeros((16,), jnp.float32)
        return carry
    lax.fori_loop(0, STRIPE // 16, fill_zer, 0)

    sl = pl.ds(s * STRIPE, STRIPE)
    pltpu.sync_copy(rs_v, sh_a.at[sl])
    pltpu.sync_copy(rs_v, sh_b.at[sl])
    plsc.subcore_barrier()

    def hist(arr, row, sh):
        def chunk_body(j, carry):
            base = s * ET + j * HC
            pltpu.sync_copy(arr.at[row, pl.ds(base, HC)], idx_v)
            pltpu.sync_copy(ones_v, sh.at[idx_v], add=True)
            return carry
        lax.fori_loop(0, HCH, chunk_body, 0)

    @pl.when(c == 0)
    def _():
        hist(e_ui, 0, sh_a)
        hist(e_ui, 1, sh_b)

    @pl.when(c == 1)
    def _():
        hist(e_iu, 0, sh_a)
        hist(e_iu, 1, sh_b)

    plsc.subcore_barrier()

    def rs_from(sh):
        pltpu.sync_copy(sh.at[sl], deg_v)

        def body(i, carry):
            rs_v[pl.ds(i * 16, 16)] = _nr_rsqrt(deg_v[pl.ds(i * 16, 16)])
            return carry
        lax.fori_loop(0, STRIPE // 16, body, 0)

    rs_from(sh_b)

    @pl.when(c == 0)
    def _():
        pltpu.sync_copy(rs_v, rs_du.at[sl])

    @pl.when(c == 1)
    def _():
        pltpu.sync_copy(rs_v, rs_di.at[sl])

    rs_from(sh_a)

    def prescale(x, ql, qh):
        for t in range(STRIPE // RCH):
            base = s * STRIPE + t * RCH
            off = jnp.minimum(base, N - RCH)
            pltpu.sync_copy(x.at[pl.ds(off, RCH)], x_v)
            rbase = off - s * STRIPE

            def grp_body(g, carry):
                rsvec = rs_v[pl.ds(rbase + g * 16, 16)]
                for rr in range(16):
                    r = g * 16 + rr
                    rsc = rsvec[rr]
                    qbl[r, pl.ds(0, Q)] = x_v[r, pl.ds(0, Q)] * rsc
                    qbl[r, pl.ds(Q, Q)] = x_v[r, pl.ds(Q, Q)] * rsc
                    qbh[r, pl.ds(0, Q)] = x_v[r, pl.ds(2 * Q, Q)] * rsc
                    qbh[r, pl.ds(Q, Q)] = x_v[r, pl.ds(3 * Q, Q)] * rsc
                return carry
            lax.fori_loop(0, RCH // 16, grp_body, 0)
            osl = pl.ds(off, RCH)
            pltpu.sync_copy(qbl, ql.at[osl])
            pltpu.sync_copy(qbh, qh.at[osl])

    @pl.when(c == 0)
    def _():
        prescale(xu, tul, tuh)

    @pl.when(c == 1)
    def _():
        prescale(xi, til, tih)


H = 2 * Q
K2C = 200
K2G = 25
K2GR = ET // (K2C * K2G)


@functools.partial(
    pl.kernel,
    out_type=jax.ShapeDtypeStruct((N, D), jnp.float32),
    mesh=_mesh,
    scratch_types=[
        pltpu.VMEM((K2C * K2G,), jnp.int32),
        pltpu.VMEM((K2C * K2G,), jnp.int32),
        pltpu.VMEM((K2C, H), jnp.float32),
        pltpu.VMEM((K2C, H), jnp.float32),
        pltpu.VMEM((STRIPE // 32, H), jnp.float32),
        pltpu.VMEM((STRIPE // 32, H), jnp.float32),
        pltpu.VMEM_SHARED((NPAD, H), jnp.float32),
        pltpu.SemaphoreType.DMA,
        pltpu.SemaphoreType.DMA,
        pltpu.SemaphoreType.DMA,
        pltpu.SemaphoreType.DMA,
    ],
    compiler_params=_sc_params,
)
def _segsum_kernel(tlo, thi, earr, a_out,
                   idxs, idxd, rows0, rows1, zer_v, bnc_v, sh,
                   semg0, semg1, sems0, sems1):
    c = lax.axis_index("c")
    s = lax.axis_index("s")

    def fill_zer(i, carry):
        zer_v[i, pl.ds(0, 16)] = jnp.zeros((16,), jnp.float32)
        zer_v[i, pl.ds(16, 16)] = jnp.zeros((16,), jnp.float32)
        return carry
    lax.fori_loop(0, STRIPE // 32, fill_zer, 0)

    def zero_shared():
        for t in range(32):
            pltpu.sync_copy(zer_v, sh.at[pl.ds(s * STRIPE + t * (STRIPE // 32),
                                               STRIPE // 32)])

    rows = (rows0, rows1)
    semg = (semg0, semg1)
    sems = (sems0, sems1)

    def accumulate(tab, earr):
        def group_body(g, carry):
            base = s * ET + g * (K2C * K2G)
            pltpu.sync_copy(earr.at[0, pl.ds(base, K2C * K2G)], idxs)
            pltpu.sync_copy(earr.at[1, pl.ds(base, K2C * K2G)], idxd)
            gd = [None] * K2G
            sd = [None] * K2G
            for k in range(K2G):
                sl = k % 2
                if k >= 2:
                    sd[k - 2].wait()
                gd[k] = pltpu.async_copy(
                    tab.at[idxs.at[pl.ds(k * K2C, K2C)]], rows[sl], semg[sl])
                if k >= 1:
                    psl = (k - 1) % 2
                    gd[k - 1].wait()
                    sd[k - 1] = pltpu.async_copy(
                        rows[psl], sh.at[idxd.at[pl.ds((k - 1) * K2C, K2C)]],
                        sems[psl], add=True)
            gd[K2G - 1].wait()
            lsl = (K2G - 1) % 2
            sd[K2G - 1] = pltpu.async_copy(
                rows[lsl], sh.at[idxd.at[pl.ds((K2G - 1) * K2C, K2C)]],
                sems[lsl], add=True)
            sd[K2G - 2].wait()
            sd[K2G - 1].wait()
            return carry
        lax.fori_loop(0, K2GR, group_body, 0)

    def writeout(out, half):
        for t in range(32):
            off = jnp.minimum(s * STRIPE + t * (STRIPE // 32), N - STRIPE // 32)
            rsl = pl.ds(off, STRIPE // 32)
            pltpu.sync_copy(sh.at[rsl], bnc_v)
            pltpu.sync_copy(bnc_v, out.at[rsl, pl.ds(half * H, H)])

    zero_shared()
    plsc.subcore_barrier()

    @pl.when(c == 0)
    def _():
        accumulate(tlo, earr)

    @pl.when(c == 1)
    def _():
        accumulate(thi, earr)

    plsc.subcore_barrier()

    @pl.when(c == 0)
    def _():
        writeout(a_out, 0)

    @pl.when(c == 1)
    def _():
        writeout(a_out, 1)


_BLK = 2000


def _combine_body(xd_ref, a_ref, rs_ref, wl_t, wi_t, xc_ref):
    xd = xd_ref[...]
    a = a_ref[...] * rs_ref[...]
    z = (jnp.dot(xd + a, wl_t[...], preferred_element_type=jnp.float32)
         + jnp.dot(xd * a, wi_t[...], preferred_element_type=jnp.float32))
    xc_ref[...] = jnp.concatenate([xd, jnp.where(z >= 0, z, 0.01 * z)], axis=-1)


def _combine(x_d, a, rs, wl_t, wi_t):
    grid = (N // _BLK,)
    row = pl.BlockSpec((_BLK, D), lambda i: (i, 0))
    col = pl.BlockSpec((_BLK, 1), lambda i: (i, 0))
    wide = pl.BlockSpec((_BLK, 2 * D), lambda i: (i, 0))
    wspec = pl.BlockSpec((D, D), lambda i: (0, 0))
    return pl.pallas_call(
        _combine_body,
        grid=grid,
        in_specs=[row, row, col, wspec, wspec],
        out_specs=wide,
        out_shape=jax.ShapeDtypeStruct((N, 2 * D), jnp.float32),
    )(x_d, a, rs, wl_t, wi_t)


@functools.partial(
    pl.kernel,
    out_type=jax.ShapeDtypeStruct((L_PAD,), jnp.float32),
    mesh=_mesh,
    scratch_types=[
        pltpu.VMEM((LW,), jnp.int32),
        pltpu.VMEM((LW,), jnp.int32),
        pltpu.VMEM((128, 2 * D), jnp.float32),
        pltpu.VMEM((128, 2 * D), jnp.float32),
        pltpu.VMEM((128, 2 * D), jnp.float32),
        pltpu.VMEM((128, 2 * D), jnp.float32),
        pltpu.VMEM((LW,), jnp.float32),
        pltpu.SemaphoreType.DMA,
        pltpu.SemaphoreType.DMA,
    ],
    compiler_params=_sc_params,
)
def _label_kernel(xcu, xci, l0, l1, y,
                  l0_v, l1_v, a_b0, b_b0, a_b1, b_b1, y_b, sem0, sem1):
    c = lax.axis_index("c")
    s = lax.axis_index("s")
    w = c * 16 + s

    pltpu.sync_copy(l0.at[pl.ds(w * LW, LW)], l0_v)
    pltpu.sync_copy(l1.at[pl.ds(w * LW, LW)], l1_v)

    lanes = lax.iota(jnp.int32, 16)

    def dots(j, a_b, b_b):
        def group_body(g, carry2):
            def lane_body(rr, vec):
                r = g * 16 + rr
                acc = a_b[r, pl.ds(0, 16)] * b_b[r, pl.ds(0, 16)]
                for q in range(1, 8):
                    acc = acc + a_b[r, pl.ds(q * 16, 16)] * b_b[r, pl.ds(q * 16, 16)]
                return jnp.where(lanes == rr, jnp.sum(acc), vec)
            vec = lax.fori_loop(0, 16, lane_body, jnp.zeros((16,), jnp.float32))
            y_b[pl.ds(j * 128 + g * 16, 16)] = vec
            return carry2
        lax.fori_loop(0, 8, group_body, 0)

    def fire(j, a_b, b_b, sem):
        sl = pl.ds(j * 128, 128)
        return (pltpu.async_copy(xcu.at[l0_v.at[sl]], a_b, sem),
                pltpu.async_copy(xci.at[l1_v.at[sl]], b_b, sem))

    def pair_body(p, carry):
        j0 = 2 * p
        da = fire(j0, a_b0, b_b0, sem0)
        db = fire(j0 + 1, a_b1, b_b1, sem1)
        for d_ in da:
            d_.wait()
        dots(j0, a_b0, b_b0)
        for d_ in db:
            d_.wait()
        dots(j0 + 1, a_b1, b_b1)
        return carry
    lax.fori_loop(0, LCH // 2, pair_body, 0)

    dt = fire(LCH - 1, a_b0, b_b0, sem0)
    for d_ in dt:
        d_.wait()
    dots(LCH - 1, a_b0, b_b0)

    pltpu.sync_copy(y_b, y.at[pl.ds(w * LW, LW)])


def kernel(n_id_user, n_id_item, edge_index_ui, edge_index_iu, edge_label_index,
           emb_user, emb_item, W_loop_ui, W_intr_ui, W_loop_iu, W_intr_iu):
    del n_id_user, n_id_item
    f32 = jnp.float32
    i32 = jnp.int32
    x_u = emb_user.astype(f32)
    x_i = emb_item.astype(f32)

    e_ui = edge_index_ui.astype(i32)
    e_iu = edge_index_iu.astype(i32)

    tul, tuh, til, tih, rs_du, rs_di = _prep_kernel(e_ui, e_iu, x_u, x_i)

    a_ui = _segsum_kernel(tul, tuh, e_ui)
    xcat_i = _combine(x_i, a_ui, rs_du[:N].reshape(N, 1),
                      W_loop_ui.T, W_intr_ui.T)
    a_iu = _segsum_kernel(til, tih, e_iu)
    xcat_u = _combine(x_u, a_iu, rs_di[:N].reshape(N, 1),
                      W_loop_iu.T, W_intr_iu.T)

    lpad = jnp.zeros((L_PAD - L,), i32)
    l0 = jnp.concatenate([edge_label_index[0].astype(i32), lpad])
    l1 = jnp.concatenate([edge_label_index[1].astype(i32), lpad])
    y = _label_kernel(xcat_u, xcat_i, l0, l1)
    return y[:L]


# speedup vs baseline: 1.3213x; 1.0112x over previous
import functools

import jax
import jax.numpy as jnp
from jax import lax
from jax.experimental import pallas as pl
from jax.experimental.pallas import tpu as pltpu
from jax.experimental.pallas import tpu_sc as plsc

N = 50000
D = 64
Q = 16
E = 800000
ET = E // 16
HC = 2000
HCH = ET // HC
NPAD = 50176
STRIPE = NPAD // 16
RCH = 224
L = 100000
L_PAD = 102400
LW = L_PAD // 32
LCH = LW // 128

_mesh = plsc.VectorSubcoreMesh(core_axis_name="c", subcore_axis_name="s",
                               num_cores=2, num_subcores=16)
_sc_params = pltpu.CompilerParams(use_tc_tiling_on_sc=False,
                                  needs_layout_passes=False)


def _nr_rsqrt(v):
    x = jnp.maximum(v, 1.0)
    i = lax.bitcast_convert_type(x, jnp.int32)
    i = jnp.int32(0x5F3759DF) - (i >> 1)
    y = lax.bitcast_convert_type(i, jnp.float32)
    for _ in range(3):
        y = y * (1.5 - 0.5 * x * y * y)
    return y


@functools.partial(
    pl.kernel,
    out_type=(tuple(jax.ShapeDtypeStruct((NPAD, 2 * Q), jnp.float32) for _ in range(4))
              + (jax.ShapeDtypeStruct((NPAD,), jnp.float32),
                 jax.ShapeDtypeStruct((NPAD,), jnp.float32))),
    mesh=_mesh,
    scratch_types=[
        pltpu.VMEM((HC,), jnp.int32),
        pltpu.VMEM((HC,), jnp.float32),
        pltpu.VMEM((STRIPE,), jnp.float32),
        pltpu.VMEM((STRIPE,), jnp.float32),
        pltpu.VMEM((RCH, D), jnp.float32),
        pltpu.VMEM((RCH, 2 * Q), jnp.float32),
        pltpu.VMEM((RCH, 2 * Q), jnp.float32),
        pltpu.VMEM_SHARED((NPAD,), jnp.float32),
        pltpu.VMEM_SHARED((NPAD,), jnp.float32),
    ],
    compiler_params=_sc_params,
)
def _prep_kernel(e_ui, e_iu, xu, xi,
                 tul, tuh, til, tih, rs_du, rs_di,
                 idx_v, ones_v, deg_v, rs_v, x_v, qbl, qbh,
                 sh_a, sh_b):
    c = lax.axis_index("c")
    s = lax.axis_index("s")

    def fill_ones(i, carry):
        ones_v[pl.ds(i * 16, 16)] = jnp.ones((16,), jnp.float32)
        return carry
    lax.fori_loop(0, HC // 16, fill_ones, 0)

    def fill_zer(i, carry):
        rs_v[pl.ds(i * 16, 16)] = jnp.zeros((16,), jnp.float32)
        return carry
    lax.fori_loop(0, STRIPE // 16, fill_zer, 0)

    sl = pl.ds(s * STRIPE, STRIPE)
    pltpu.sync_copy(rs_v, sh_a.at[sl])
    pltpu.sync_copy(rs_v, sh_b.at[sl])
    plsc.subcore_barrier()

    def hist(arr, row, sh):
        def chunk_body(j, carry):
            base = s * ET + j * HC
            pltpu.sync_copy(arr.at[row, pl.ds(base, HC)], idx_v)
            pltpu.sync_copy(ones_v, sh.at[idx_v], add=True)
            return carry
        lax.fori_loop(0, HCH, chunk_body, 0)

    @pl.when(c == 0)
    def _():
        hist(e_ui, 0, sh_a)
        hist(e_ui, 1, sh_b)

    @pl.when(c == 1)
    def _():
        hist(e_iu, 0, sh_a)
        hist(e_iu, 1, sh_b)

    plsc.subcore_barrier()

    def rs_from(sh):
        pltpu.sync_copy(sh.at[sl], deg_v)

        def body(i, carry):
            rs_v[pl.ds(i * 16, 16)] = _nr_rsqrt(deg_v[pl.ds(i * 16, 16)])
            return carry
        lax.fori_loop(0, STRIPE // 16, body, 0)

    rs_from(sh_b)

    @pl.when(c == 0)
    def _():
        pltpu.sync_copy(rs_v, rs_du.at[sl])

    @pl.when(c == 1)
    def _():
        pltpu.sync_copy(rs_v, rs_di.at[sl])

    rs_from(sh_a)

    def prescale(x, ql, qh):
        for t in range(STRIPE // RCH):
            base = s * STRIPE + t * RCH
            off = jnp.minimum(base, N - RCH)
            pltpu.sync_copy(x.at[pl.ds(off, RCH)], x_v)
            rbase = off - s * STRIPE

            def grp_body(g, carry):
                rsvec = rs_v[pl.ds(rbase + g * 16, 16)]
                for rr in range(16):
                    r = g * 16 + rr
                    rsc = rsvec[rr]
                    qbl[r, pl.ds(0, Q)] = x_v[r, pl.ds(0, Q)] * rsc
                    qbl[r, pl.ds(Q, Q)] = x_v[r, pl.ds(Q, Q)] * rsc
                    qbh[r, pl.ds(0, Q)] = x_v[r, pl.ds(2 * Q, Q)] * rsc
                    qbh[r, pl.ds(Q, Q)] = x_v[r, pl.ds(3 * Q, Q)] * rsc
                return carry
            lax.fori_loop(0, RCH // 16, grp_body, 0)
            osl = pl.ds(off, RCH)
            pltpu.sync_copy(qbl, ql.at[osl])
            pltpu.sync_copy(qbh, qh.at[osl])

    @pl.when(c == 0)
    def _():
        prescale(xu, tul, tuh)

    @pl.when(c == 1)
    def _():
        prescale(xi, til, tih)


H = 2 * Q
K2C = 200
K2G = 10
K2GR = ET // (K2C * K2G)


@functools.partial(
    pl.kernel,
    out_type=(jax.ShapeDtypeStruct((N, D), jnp.float32),
              jax.ShapeDtypeStruct((N, D), jnp.float32)),
    mesh=_mesh,
    scratch_types=[
        pltpu.VMEM((K2C * K2G,), jnp.int32),
        pltpu.VMEM((K2C * K2G,), jnp.int32),
        pltpu.VMEM((K2C, H), jnp.float32),
        pltpu.VMEM((K2C, H), jnp.float32),
        pltpu.VMEM((K2C, H), jnp.float32),
        pltpu.VMEM((STRIPE // 32, H), jnp.float32),
        pltpu.VMEM((STRIPE // 32, H), jnp.float32),
        pltpu.VMEM_SHARED((NPAD, H), jnp.float32),
        pltpu.SemaphoreType.DMA,
        pltpu.SemaphoreType.DMA,
        pltpu.SemaphoreType.DMA,
        pltpu.SemaphoreType.DMA,
    ],
    compiler_params=_sc_params,
)
def _segsum_kernel(tul, tuh, til, tih,
                   e_ui, e_iu,
                   a_ui, a_iu,
                   idxs, idxd, rows0, rows1, rows2, zer_v, bnc_v, sh,
                   semg0, semg1, sems0, sems1):
    c = lax.axis_index("c")
    s = lax.axis_index("s")

    def fill_zer(i, carry):
        zer_v[i, pl.ds(0, 16)] = jnp.zeros((16,), jnp.float32)
        zer_v[i, pl.ds(16, 16)] = jnp.zeros((16,), jnp.float32)
        return carry
    lax.fori_loop(0, STRIPE // 32, fill_zer, 0)

    def zero_shared():
        for t in range(32):
            pltpu.sync_copy(zer_v, sh.at[pl.ds(s * STRIPE + t * (STRIPE // 32),
                                               STRIPE // 32)])

    rows = (rows0, rows1, rows2)
    semg = (semg0, semg1, semg0)
    sems = (sems0, sems1, sems0)

    def accumulate(tab, earr):
        def group_body(g, carry):
            base = s * ET + g * (K2C * K2G)
            pltpu.sync_copy(earr.at[0, pl.ds(base, K2C * K2G)], idxs)
            pltpu.sync_copy(earr.at[1, pl.ds(base, K2C * K2G)], idxd)
            gd = [None] * K2G
            sd = [None] * K2G
            for k in range(K2G):
                sl = k % 3
                if k >= 3:
                    sd[k - 3].wait()
                gd[k] = pltpu.async_copy(
                    tab.at[idxs.at[pl.ds(k * K2C, K2C)]], rows[sl], semg[sl])
                if k >= 1:
                    psl = (k - 1) % 3
                    gd[k - 1].wait()
                    sd[k - 1] = pltpu.async_copy(
                        rows[psl], sh.at[idxd.at[pl.ds((k - 1) * K2C, K2C)]],
                        sems[psl], add=True)
            gd[K2G - 1].wait()
            lsl = (K2G - 1) % 3
            sd[K2G - 1] = pltpu.async_copy(
                rows[lsl], sh.at[idxd.at[pl.ds((K2G - 1) * K2C, K2C)]],
                sems[lsl], add=True)
            sd[K2G - 3].wait()
            sd[K2G - 2].wait()
            sd[K2G - 1].wait()
            return carry
        lax.fori_loop(0, K2GR, group_body, 0)

    def writeout(out, half):
        for t in range(32):
            off = jnp.minimum(s * STRIPE + t * (STRIPE // 32), N - STRIPE // 32)
            rsl = pl.ds(off, STRIPE // 32)
            pltpu.sync_copy(sh.at[rsl], bnc_v)
            pltpu.sync_copy(bnc_v, out.at[rsl, pl.ds(half * H, H)])

    phases = (
        ((tul, e_ui, a_ui), (tuh, e_ui, a_ui)),
        ((til, e_iu, a_iu), (tih, e_iu, a_iu)),
    )
    for (tb0, e0, o0), (tb1, e1, o1) in phases:
        zero_shared()
        plsc.subcore_barrier()

        @pl.when(c == 0)
        def _():
            accumulate(tb0, e0)

        @pl.when(c == 1)
        def _():
            accumulate(tb1, e1)

        plsc.subcore_barrier()

        @pl.when(c == 0)
        def _():
            writeout(o0, 0)

        @pl.when(c == 1)
        def _():
            writeout(o1, 1)

        plsc.subcore_barrier()


_BLK = 2000


def _combine_body(xu_ref, xi_ref, aui_ref, aiu_ref, rdu_ref, rdi_ref,
                  wlu_t, wiu_t, wli_t, wii_t,
                  xcu_ref, xci_ref):
    def cell(xd, a_ref, rs, wl_t, wi_t):
        a = a_ref[...] * rs
        z = (jnp.dot(xd + a, wl_t, preferred_element_type=jnp.float32)
             + jnp.dot(xd * a, wi_t, preferred_element_type=jnp.float32))
        return jnp.where(z >= 0, z, 0.01 * z)

    xi_ = xi_ref[...]
    xu_ = xu_ref[...]
    xci_ref[...] = jnp.concatenate(
        [xi_, cell(xi_, aui_ref, rdu_ref[...], wlu_t[...], wiu_t[...])], axis=-1)
    xcu_ref[...] = jnp.concatenate(
        [xu_, cell(xu_, aiu_ref, rdi_ref[...], wli_t[...], wii_t[...])], axis=-1)


def _combine(x_u, x_i, a_ui, a_iu, rdu, rdi, wlu_t, wiu_t, wli_t, wii_t):
    grid = (N // _BLK,)
    row = pl.BlockSpec((_BLK, D), lambda i: (i, 0))
    col = pl.BlockSpec((_BLK, 1), lambda i: (i, 0))
    wide = pl.BlockSpec((_BLK, 2 * D), lambda i: (i, 0))
    wspec = pl.BlockSpec((D, D), lambda i: (0, 0))
    return pl.pallas_call(
        _combine_body,
        grid=grid,
        in_specs=[row, row, row, row, col, col, wspec, wspec, wspec, wspec],
        out_specs=[wide, wide],
        out_shape=(jax.ShapeDtypeStruct((N, 2 * D), jnp.float32),
                   jax.ShapeDtypeStruct((N, 2 * D), jnp.float32)),
    )(x_u, x_i, a_ui, a_iu, rdu, rdi, wlu_t, wiu_t, wli_t, wii_t)


@functools.partial(
    pl.kernel,
    out_type=jax.ShapeDtypeStruct((L_PAD,), jnp.float32),
    mesh=_mesh,
    scratch_types=[
        pltpu.VMEM((LW,), jnp.int32),
        pltpu.VMEM((LW,), jnp.int32),
        pltpu.VMEM((128, 2 * D), jnp.float32),
        pltpu.VMEM((128, 2 * D), jnp.float32),
        pltpu.VMEM((128, 2 * D), jnp.float32),
        pltpu.VMEM((128, 2 * D), jnp.float32),
        pltpu.VMEM((LW,), jnp.float32),
        pltpu.SemaphoreType.DMA,
        pltpu.SemaphoreType.DMA,
    ],
    compiler_params=_sc_params,
)
def _label_kernel(xcu, xci, l0, l1, y,
                  l0_v, l1_v, a_b0, b_b0, a_b1, b_b1, y_b, sem0, sem1):
    c = lax.axis_index("c")
    s = lax.axis_index("s")
    w = c * 16 + s

    pltpu.sync_copy(l0.at[pl.ds(w * LW, LW)], l0_v)
    pltpu.sync_copy(l1.at[pl.ds(w * LW, LW)], l1_v)

    lanes = lax.iota(jnp.int32, 16)

    def dots(j, a_b, b_b):
        def group_body(g, carry2):
            def lane_body(rr, vec):
                r = g * 16 + rr
                acc = a_b[r, pl.ds(0, 16)] * b_b[r, pl.ds(0, 16)]
                for q in range(1, 8):
                    acc = acc + a_b[r, pl.ds(q * 16, 16)] * b_b[r, pl.ds(q * 16, 16)]
                return jnp.where(lanes == rr, jnp.sum(acc), vec)
            vec = lax.fori_loop(0, 16, lane_body, jnp.zeros((16,), jnp.float32))
            y_b[pl.ds(j * 128 + g * 16, 16)] = vec
            return carry2
        lax.fori_loop(0, 8, group_body, 0)

    def fire(j, a_b, b_b, sem):
        sl = pl.ds(j * 128, 128)
        return (pltpu.async_copy(xcu.at[l0_v.at[sl]], a_b, sem),
                pltpu.async_copy(xci.at[l1_v.at[sl]], b_b, sem))

    def pair_body(p, carry):
        j0 = 2 * p
        da = fire(j0, a_b0, b_b0, sem0)
        db = fire(j0 + 1, a_b1, b_b1, sem1)
        for d_ in da:
            d_.wait()
        dots(j0, a_b0, b_b0)
        for d_ in db:
            d_.wait()
        dots(j0 + 1, a_b1, b_b1)
        return carry
    lax.fori_loop(0, LCH // 2, pair_body, 0)

    dt = fire(LCH - 1, a_b0, b_b0, sem0)
    for d_ in dt:
        d_.wait()
    dots(LCH - 1, a_b0, b_b0)

    pltpu.sync_copy(y_b, y.at[pl.ds(w * LW, LW)])


def kernel(n_id_user, n_id_item, edge_index_ui, edge_index_iu, edge_label_index,
           emb_user, emb_item, W_loop_ui, W_intr_ui, W_loop_iu, W_intr_iu):
    del n_id_user, n_id_item
    f32 = jnp.float32
    i32 = jnp.int32
    x_u = emb_user.astype(f32)
    x_i = emb_item.astype(f32)

    e_ui = edge_index_ui.astype(i32)
    e_iu = edge_index_iu.astype(i32)

    tul, tuh, til, tih, rs_du, rs_di = _prep_kernel(e_ui, e_iu, x_u, x_i)

    a_ui, a_iu = _segsum_kernel(tul, tuh, til, tih, e_ui, e_iu)

    xcat_u, xcat_i = _combine(
        x_u, x_i, a_ui, a_iu,
        rs_du[:N].reshape(N, 1), rs_di[:N].reshape(N, 1),
        W_loop_ui.T, W_intr_ui.T, W_loop_iu.T, W_intr_iu.T)

    lpad = jnp.zeros((L_PAD - L,), i32)
    l0 = jnp.concatenate([edge_label_index[0].astype(i32), lpad])
    l1 = jnp.concatenate([edge_label_index[1].astype(i32), lpad])
    y = _label_kernel(xcat_u, xcat_i, l0, l1)
    return y[:L]
